# baseline (speedup 1.0000x reference)
import jax
import jax.numpy as jnp
from jax.experimental import pallas as pl

NUM_GROUP = 512
GROUP_SIZE = 32


def _fps(xyz, num_group):
    B, N, _ = xyz.shape

    def step(carry, _):
        distances, farthest = carry
        centroid = jnp.take_along_axis(xyz, farthest[:, None, None], axis=1)
        d = jnp.sum((xyz - centroid) ** 2, axis=-1)
        distances = jnp.minimum(distances, d)
        nxt = jnp.argmax(distances, axis=-1).astype(jnp.int32)
        return (distances, nxt), farthest

    init = (jnp.full((B, N), 1e10, dtype=xyz.dtype), jnp.zeros((B,), dtype=jnp.int32))
    _, idxs = jax.lax.scan(step, init, None, length=num_group)
    return jnp.transpose(idxs, (1, 0))


def _index_points(points, idx):
    return jax.vmap(lambda p, i: p[i])(points, idx)


def _sub_kernel(nbr_ref, ctr_ref, out_ref):
    out_ref[...] = nbr_ref[...] - ctr_ref[...]


def kernel(xyz):
    B, N, _ = xyz.shape
    center_idx = _fps(xyz, NUM_GROUP)
    center = _index_points(xyz, center_idx)
    d2 = (jnp.sum(center ** 2, axis=-1)[:, :, None]
          + jnp.sum(xyz ** 2, axis=-1)[:, None, :]
          - 2.0 * jnp.einsum('bgd,bnd->bgn', center, xyz))
    _, idx = jax.lax.top_k(-d2, GROUP_SIZE)
    nbr = _index_points(xyz, idx)
    nbr2 = nbr.reshape(B, NUM_GROUP * GROUP_SIZE, 3)
    ctr2 = jnp.repeat(center, GROUP_SIZE, axis=1)
    out = pl.pallas_call(
        _sub_kernel,
        out_shape=jax.ShapeDtypeStruct((B, NUM_GROUP * GROUP_SIZE, 3), xyz.dtype),
        grid=(B,),
        in_specs=[
            pl.BlockSpec((1, NUM_GROUP * GROUP_SIZE, 3), lambda b: (b, 0, 0)),
            pl.BlockSpec((1, NUM_GROUP * GROUP_SIZE, 3), lambda b: (b, 0, 0)),
        ],
        out_specs=pl.BlockSpec((1, NUM_GROUP * GROUP_SIZE, 3), lambda b: (b, 0, 0)),
    )(nbr2, ctr2)
    neighborhood = out.reshape(B, NUM_GROUP, GROUP_SIZE, 3)
    return (neighborhood, center, idx)


# speedup vs baseline: 1.4398x; 1.4398x over previous
import functools

import jax
import jax.numpy as jnp
from jax.experimental import pallas as pl
from jax.experimental.pallas import tpu as pltpu

NUM_GROUP = 512
GROUP_SIZE = 32
B = 32
N = 8192


def _fma_sq_add(a, c):
    t = a * 4097.0
    u = t - a
    hi = t - u
    lo = a - hi
    p = a * a
    hh = hi * hi
    e = ((hh - p) + (hi * lo) * 2.0) + lo * lo
    s = c + p
    bv = s - c
    err = (c - (s - bv)) + (p - bv)
    return s + (err + e)


def _ref_dist(dx, dy, dz):
    return _fma_sq_add(dz, _fma_sq_add(dy, dx * dx))


def _fps_kernel(xt_ref, idx_ref, ctr_ref, dist_ref):
    x = xt_ref[0]
    y = xt_ref[1]
    z = xt_ref[2]
    dist_ref[...] = jnp.full((B, N), 1e10, jnp.float32)
    iota = (jax.lax.broadcasted_iota(jnp.int32, (B, N), 1)
            + jax.lax.broadcasted_iota(jnp.int32, (B, N), 0) * 0)
    giota = (jax.lax.broadcasted_iota(jnp.int32, (B, NUM_GROUP), 1)
             + jax.lax.broadcasted_iota(jnp.int32, (B, NUM_GROUP), 0) * 0)

    def step(t, carry):
        far, idx_acc, cx_acc, cy_acc, cz_acc = carry
        onehot = iota == far
        cx = jnp.sum(jnp.where(onehot, x, 0.0), axis=1, keepdims=True)
        cy = jnp.sum(jnp.where(onehot, y, 0.0), axis=1, keepdims=True)
        cz = jnp.sum(jnp.where(onehot, z, 0.0), axis=1, keepdims=True)
        slot = giota == t
        idx_acc = jnp.where(slot, far, idx_acc)
        cx_acc = jnp.where(slot, cx, cx_acc)
        cy_acc = jnp.where(slot, cy, cy_acc)
        cz_acc = jnp.where(slot, cz, cz_acc)
        d = _ref_dist(x - cx, y - cy, z - cz)
        nd = jnp.minimum(dist_ref[...], d)
        dist_ref[...] = nd
        m = jnp.max(nd, axis=1, keepdims=True)
        nxt = jnp.min(jnp.where(nd == m, iota, N), axis=1, keepdims=True)
        return (nxt, idx_acc, cx_acc, cy_acc, cz_acc)

    idx_ref[...] = jnp.zeros((B, NUM_GROUP), jnp.int32)
    ctr_ref[...] = jnp.zeros((3, B, NUM_GROUP), jnp.float32)
    far0 = jnp.min(iota, axis=1, keepdims=True)
    init = (far0, idx_ref[...], ctr_ref[0], ctr_ref[1], ctr_ref[2])
    _, idx_acc, cx_acc, cy_acc, cz_acc = jax.lax.fori_loop(
        0, NUM_GROUP, step, init)
    idx_ref[...] = idx_acc
    ctr_ref[0] = cx_acc
    ctr_ref[1] = cy_acc
    ctr_ref[2] = cz_acc


def _fps_pallas(xt):
    return pl.pallas_call(
        _fps_kernel,
        out_shape=[
            jax.ShapeDtypeStruct((B, NUM_GROUP), jnp.int32),
            jax.ShapeDtypeStruct((3, B, NUM_GROUP), jnp.float32),
        ],
        scratch_shapes=[pltpu.VMEM((B, N), jnp.float32)],
    )(xt)


def _index_points(points, idx):
    return jax.vmap(lambda p, i: p[i])(points, idx)


def kernel(xyz):
    xt = jnp.transpose(xyz, (2, 0, 1))
    center_idx, ctr = _fps_pallas(xt)
    center = jnp.transpose(ctr, (1, 2, 0))
    d2 = (jnp.sum(center ** 2, axis=-1)[:, :, None]
          + jnp.sum(xyz ** 2, axis=-1)[:, None, :]
          - 2.0 * jnp.einsum('bgd,bnd->bgn', center, xyz))
    _, idx = jax.lax.top_k(-d2, GROUP_SIZE)
    neighborhood = _index_points(xyz, idx) - center[:, :, None, :]
    return (neighborhood, center, idx)


# speedup vs baseline: 3.9013x; 2.7096x over previous
import functools

import jax
import jax.numpy as jnp
from jax.experimental import pallas as pl
from jax.experimental.pallas import tpu as pltpu

NUM_GROUP = 512
GROUP_SIZE = 32
B = 32
N = 8192


def _fma_sq_add(a, c):
    t = a * 4097.0
    u = t - a
    hi = t - u
    lo = a - hi
    p = a * a
    hh = hi * hi
    e = ((hh - p) + (hi * lo) * 2.0) + lo * lo
    s = c + p
    bv = s - c
    err = (c - (s - bv)) + (p - bv)
    return s + (err + e)


def _ref_dist(dx, dy, dz):
    return _fma_sq_add(dz, _fma_sq_add(dy, dx * dx))


def _fps_kernel(xt_ref, idx_ref, ctr_ref, dist_ref):
    x = xt_ref[0]
    y = xt_ref[1]
    z = xt_ref[2]
    dist_ref[...] = jnp.full((B, N), 1e10, jnp.float32)
    iota = (jax.lax.broadcasted_iota(jnp.int32, (B, N), 1)
            + jax.lax.broadcasted_iota(jnp.int32, (B, N), 0) * 0)
    giota = (jax.lax.broadcasted_iota(jnp.int32, (B, NUM_GROUP), 1)
             + jax.lax.broadcasted_iota(jnp.int32, (B, NUM_GROUP), 0) * 0)

    def step(t, carry):
        far, idx_acc, cx_acc, cy_acc, cz_acc = carry
        onehot = iota == far
        cx = jnp.sum(jnp.where(onehot, x, 0.0), axis=1, keepdims=True)
        cy = jnp.sum(jnp.where(onehot, y, 0.0), axis=1, keepdims=True)
        cz = jnp.sum(jnp.where(onehot, z, 0.0), axis=1, keepdims=True)
        slot = giota == t
        idx_acc = jnp.where(slot, far, idx_acc)
        cx_acc = jnp.where(slot, cx, cx_acc)
        cy_acc = jnp.where(slot, cy, cy_acc)
        cz_acc = jnp.where(slot, cz, cz_acc)
        d = _ref_dist(x - cx, y - cy, z - cz)
        nd = jnp.minimum(dist_ref[...], d)
        dist_ref[...] = nd
        m = jnp.max(nd, axis=1, keepdims=True)
        nxt = jnp.min(jnp.where(nd == m, iota, N), axis=1, keepdims=True)
        return (nxt, idx_acc, cx_acc, cy_acc, cz_acc)

    idx_ref[...] = jnp.zeros((B, NUM_GROUP), jnp.int32)
    ctr_ref[...] = jnp.zeros((3, B, NUM_GROUP), jnp.float32)
    far0 = jnp.min(iota, axis=1, keepdims=True)
    init = (far0, idx_ref[...], ctr_ref[0], ctr_ref[1], ctr_ref[2])
    _, idx_acc, cx_acc, cy_acc, cz_acc = jax.lax.fori_loop(
        0, NUM_GROUP, step, init)
    idx_ref[...] = idx_acc
    ctr_ref[0] = cx_acc
    ctr_ref[1] = cy_acc
    ctr_ref[2] = cz_acc


def _fps_pallas(xt):
    return pl.pallas_call(
        _fps_kernel,
        out_shape=[
            jax.ShapeDtypeStruct((B, NUM_GROUP), jnp.int32),
            jax.ShapeDtypeStruct((3, B, NUM_GROUP), jnp.float32),
        ],
        scratch_shapes=[pltpu.VMEM((B, N), jnp.float32)],
    )(xt)


def _index_points(points, idx):
    return jax.vmap(lambda p, i: p[i])(points, idx)


GT = 64


def _knn_kernel(ct_ref, xt_ref, idx_ref, d2_ref, iota_ref):
    c = ct_ref[0]
    x3 = xt_ref[0]
    x = x3[0:1, :]
    y = x3[1:2, :]
    z = x3[2:3, :]
    x2 = (x * x + y * y) + z * z
    cx = c[:, 0:1]
    cy = c[:, 1:2]
    cz = c[:, 2:3]
    c2 = (cx * cx + cy * cy) + cz * cz
    e = jax.lax.dot_general(c, x3, (((1,), (0,)), ((), ())),
                            preferred_element_type=jnp.float32)
    d2_ref[...] = (c2 + x2) - 2.0 * e
    iota_ref[...] = (jax.lax.broadcasted_iota(jnp.int32, (GT, N), 1)
                     + jax.lax.broadcasted_iota(jnp.int32, (GT, N), 0) * 0)
    kiota = (jax.lax.broadcasted_iota(jnp.int32, (GT, GROUP_SIZE), 1)
             + jax.lax.broadcasted_iota(jnp.int32, (GT, GROUP_SIZE), 0) * 0)
    idx_ref[0] = jnp.zeros((GT, GROUP_SIZE), jnp.int32)
    idx0 = idx_ref[0]

    def round_k(kk, idx_acc):
        dv = d2_ref[...]
        io = iota_ref[...]
        m = jnp.min(dv, axis=1, keepdims=True)
        cand = jnp.where(dv == m, io, N)
        ai = jnp.min(cand, axis=1, keepdims=True)
        d2_ref[...] = jnp.where(io == ai, jnp.inf, dv)
        return jnp.where(kiota == kk, ai, idx_acc)

    idx_ref[0] = jax.lax.fori_loop(0, GROUP_SIZE, round_k, idx0)


def _knn_pallas(center, xtb):
    return pl.pallas_call(
        _knn_kernel,
        out_shape=jax.ShapeDtypeStruct((B, NUM_GROUP, GROUP_SIZE), jnp.int32),
        grid=(B, NUM_GROUP // GT),
        in_specs=[
            pl.BlockSpec((1, GT, 3), lambda b, g: (b, g, 0)),
            pl.BlockSpec((1, 3, N), lambda b, g: (b, 0, 0)),
        ],
        out_specs=pl.BlockSpec((1, GT, GROUP_SIZE), lambda b, g: (b, g, 0)),
        scratch_shapes=[pltpu.VMEM((GT, N), jnp.float32),
                        pltpu.VMEM((GT, N), jnp.int32)],
    )(center, xtb)


def kernel(xyz):
    xt = jnp.transpose(xyz, (2, 0, 1))
    center_idx, ctr = _fps_pallas(xt)
    center = jnp.transpose(ctr, (1, 2, 0))
    xtb = jnp.transpose(xyz, (0, 2, 1))
    idx = _knn_pallas(center, xtb)
    neighborhood = _index_points(xyz, idx) - center[:, :, None, :]
    return (neighborhood, center, idx)


# speedup vs baseline: 7.8288x; 2.0067x over previous
import functools

import functools

import jax
import jax.numpy as jnp
from jax import lax
from jax.experimental import pallas as pl
from jax.experimental.pallas import tpu as pltpu
from jax.experimental.pallas import tpu_sc as plsc

NUM_GROUP = 512
GROUP_SIZE = 32
B = 32
N = 8192


def _fma_sq_add(a, c):
    t = a * 4097.0
    u = t - a
    hi = t - u
    lo = a - hi
    p = a * a
    hh = hi * hi
    e = ((hh - p) + (hi * lo) * 2.0) + lo * lo
    s = c + p
    bv = s - c
    err = (c - (s - bv)) + (p - bv)
    return s + (err + e)


def _ref_dist(dx, dy, dz):
    return _fma_sq_add(dz, _fma_sq_add(dy, dx * dx))


def _fps_kernel(xt_ref, idx_ref, ctr_ref, dist_ref):
    x = xt_ref[0]
    y = xt_ref[1]
    z = xt_ref[2]
    dist_ref[...] = jnp.full((B, N), 1e10, jnp.float32)
    iota = (jax.lax.broadcasted_iota(jnp.int32, (B, N), 1)
            + jax.lax.broadcasted_iota(jnp.int32, (B, N), 0) * 0)
    giota = (jax.lax.broadcasted_iota(jnp.int32, (B, NUM_GROUP), 1)
             + jax.lax.broadcasted_iota(jnp.int32, (B, NUM_GROUP), 0) * 0)

    def step(t, carry):
        far, idx_acc, cx_acc, cy_acc, cz_acc = carry
        onehot = iota == far
        cx = jnp.sum(jnp.where(onehot, x, 0.0), axis=1, keepdims=True)
        cy = jnp.sum(jnp.where(onehot, y, 0.0), axis=1, keepdims=True)
        cz = jnp.sum(jnp.where(onehot, z, 0.0), axis=1, keepdims=True)
        slot = giota == t
        idx_acc = jnp.where(slot, far, idx_acc)
        cx_acc = jnp.where(slot, cx, cx_acc)
        cy_acc = jnp.where(slot, cy, cy_acc)
        cz_acc = jnp.where(slot, cz, cz_acc)
        d = _ref_dist(x - cx, y - cy, z - cz)
        nd = jnp.minimum(dist_ref[...], d)
        dist_ref[...] = nd
        m = jnp.max(nd, axis=1, keepdims=True)
        nxt = jnp.min(jnp.where(nd == m, iota, N), axis=1, keepdims=True)
        return (nxt, idx_acc, cx_acc, cy_acc, cz_acc)

    idx_ref[...] = jnp.zeros((B, NUM_GROUP), jnp.int32)
    ctr_ref[...] = jnp.zeros((3, B, NUM_GROUP), jnp.float32)
    far0 = jnp.min(iota, axis=1, keepdims=True)
    init = (far0, idx_ref[...], ctr_ref[0], ctr_ref[1], ctr_ref[2])
    _, idx_acc, cx_acc, cy_acc, cz_acc = jax.lax.fori_loop(
        0, NUM_GROUP, step, init)
    idx_ref[...] = idx_acc
    ctr_ref[0] = cx_acc
    ctr_ref[1] = cy_acc
    ctr_ref[2] = cz_acc


def _fps_pallas(xt):
    return pl.pallas_call(
        _fps_kernel,
        out_shape=[
            jax.ShapeDtypeStruct((B, NUM_GROUP), jnp.int32),
            jax.ShapeDtypeStruct((3, B, NUM_GROUP), jnp.float32),
        ],
        scratch_shapes=[pltpu.VMEM((B, N), jnp.float32)],
    )(xt)


def _index_points(points, idx):
    return jax.vmap(lambda p, i: p[i])(points, idx)


GT = 64


def _knn_kernel(ct_ref, xt_ref, idx_ref, d2_ref, iota_ref):
    c = ct_ref[0]
    x3 = xt_ref[0]
    x = x3[0:1, :]
    y = x3[1:2, :]
    z = x3[2:3, :]
    x2 = (x * x + y * y) + z * z
    cx = c[:, 0:1]
    cy = c[:, 1:2]
    cz = c[:, 2:3]
    c2 = (cx * cx + cy * cy) + cz * cz
    e = jax.lax.dot_general(c, x3, (((1,), (0,)), ((), ())),
                            preferred_element_type=jnp.float32)
    d2_ref[...] = (c2 + x2) - 2.0 * e
    iota_ref[...] = (jax.lax.broadcasted_iota(jnp.int32, (GT, N), 1)
                     + jax.lax.broadcasted_iota(jnp.int32, (GT, N), 0) * 0)
    kiota = (jax.lax.broadcasted_iota(jnp.int32, (GT, GROUP_SIZE), 1)
             + jax.lax.broadcasted_iota(jnp.int32, (GT, GROUP_SIZE), 0) * 0)
    idx_ref[0] = jnp.zeros((GT, GROUP_SIZE), jnp.int32)
    idx0 = idx_ref[0]

    def round_k(kk, idx_acc):
        dv = d2_ref[...]
        io = iota_ref[...]
        m = jnp.min(dv, axis=1, keepdims=True)
        cand = jnp.where(dv == m, io, N)
        ai = jnp.min(cand, axis=1, keepdims=True)
        d2_ref[...] = jnp.where(io == ai, jnp.inf, dv)
        return jnp.where(kiota == kk, ai, idx_acc)

    idx_ref[0] = jax.lax.fori_loop(0, GROUP_SIZE, round_k, idx0)


def _knn_pallas(center, xtb):
    return pl.pallas_call(
        _knn_kernel,
        out_shape=jax.ShapeDtypeStruct((B, NUM_GROUP, GROUP_SIZE), jnp.int32),
        grid=(B, NUM_GROUP // GT),
        in_specs=[
            pl.BlockSpec((1, GT, 3), lambda b, g: (b, g, 0)),
            pl.BlockSpec((1, 3, N), lambda b, g: (b, 0, 0)),
        ],
        out_specs=pl.BlockSpec((1, GT, GROUP_SIZE), lambda b, g: (b, g, 0)),
        scratch_shapes=[pltpu.VMEM((GT, N), jnp.float32),
                        pltpu.VMEM((GT, N), jnp.int32)],
    )(center, xtb)


_SC_CORES = 2
_SC_SUBCORES = 16
_FLAT = NUM_GROUP * GROUP_SIZE


def _gather_sc_kernel(xt_hbm, idx_hbm, ctr_hbm, out_hbm, xv, iv, cv, ov):
    wid = lax.axis_index("s") * _SC_CORES + lax.axis_index("c")
    for c in range(3):
        pltpu.sync_copy(xt_hbm.at[pl.ds((c * B + wid) * N, N)],
                        xv.at[pl.ds(c * N, N)])
        pltpu.sync_copy(ctr_hbm.at[pl.ds((c * B + wid) * NUM_GROUP, NUM_GROUP)],
                        cv.at[pl.ds(c * NUM_GROUP, NUM_GROUP)])
    pltpu.sync_copy(idx_hbm.at[pl.ds(wid * _FLAT, _FLAT)], iv)
    lanes = lax.iota(jnp.int32, 16)

    def chunk(j, carry):
        ii = iv[pl.ds(j * 16, 16)]
        g = j // 2
        gsplat = jnp.zeros((16,), jnp.int32) + g
        pos0 = lanes * 3 + j * 48
        for c in range(3):
            cval = plsc.load_gather(cv, [gsplat + c * NUM_GROUP])
            vals = plsc.load_gather(xv, [ii + c * N])
            plsc.store_scatter(ov, [pos0 + c], vals - cval)
        return carry

    lax.fori_loop(0, _FLAT // 16, chunk, 0)
    pltpu.sync_copy(ov, out_hbm.at[pl.ds(wid * (_FLAT * 3), _FLAT * 3)])


@jax.jit
def _gather_sc(xt_flat, idx_flat, ctr_flat):
    mesh = plsc.VectorSubcoreMesh(core_axis_name="c", subcore_axis_name="s")
    return pl.kernel(
        _gather_sc_kernel,
        mesh=mesh,
        compiler_params=pltpu.CompilerParams(needs_layout_passes=False),
        out_type=jax.ShapeDtypeStruct((B * _FLAT * 3,), jnp.float32),
        scratch_types=[
            pltpu.VMEM((3 * N,), jnp.float32),
            pltpu.VMEM((_FLAT,), jnp.int32),
            pltpu.VMEM((3 * NUM_GROUP,), jnp.float32),
            pltpu.VMEM((_FLAT * 3,), jnp.float32),
        ],
    )(xt_flat, idx_flat, ctr_flat)


def kernel(xyz):
    xt = jnp.transpose(xyz, (2, 0, 1))
    center_idx, ctr = _fps_pallas(xt)
    center = jnp.transpose(ctr, (1, 2, 0))
    xtb = jnp.transpose(xyz, (0, 2, 1))
    idx = _knn_pallas(center, xtb)
    flat = _gather_sc(xt.reshape(-1), idx.reshape(-1), ctr.reshape(-1))
    neighborhood = flat.reshape(B, NUM_GROUP, GROUP_SIZE, 3)
    return (neighborhood, center, idx)
